# baseline (speedup 1.0000x reference)
import numpy as np
import jax
import jax.numpy as jnp
from jax.experimental import pallas as pl
from jax.experimental.pallas import tpu as pltpu


def _cfull(shape):
    n = len(shape)
    return pl.BlockSpec(tuple(shape), lambda i, _n=n: (0,) * _n)


def kernel(x, w_conv1, b_conv1, w_conv2, b_conv2,
           w_fc1, b_fc1, w_fc2, b_fc2, w_fc3, b_fc3):
    f32 = jnp.float32
    x = x.astype(f32)
    B, cin, H, W = x.shape
    cmid = w_conv1.shape[0]

    pad = 2
    Hp, Wp = H + 2 * pad, W + 2 * pad
    H1, W1 = Hp - 1, Wp - 1
    H1p, W1p = H1 // 2, W1 // 2
    H2, W2 = H1p - 1, W1p - 1
    H2p, W2p = H2 // 2, W2 // 2
    feat = cmid * H2p * W2p
    n_h1, n_h2, n_out = w_fc1.shape[1], w_fc2.shape[1], w_fc3.shape[1]

    CB1 = B * Hp
    M1 = cmid * CB1
    SC2 = 16
    M2 = cmid * SC2
    SLAB = 8
    n_slab = cmid * H2p
    assert B * H1p <= SC2 and B <= SLAB

    NS = 4
    G = 4
    KB = feat // (G * NS)
    assert feat % (G * NS) == 0 and KB % 128 == 0

    Q = np.zeros((2, CB1, CB1), np.float32)
    vrow1 = np.zeros((CB1,), np.float32)
    for b in range(B):
        for h in range(H1):
            for dy in range(2):
                Q[dy, b * Hp + h, b * Hp + h + dy] = 1.0
            vrow1[b * Hp + h] = 1.0
    P = np.zeros((2, SC2, CB1), np.float32)
    v2row = np.zeros((SC2,), np.float32)
    for b in range(B):
        for h2 in range(H2):
            for dy in range(2):
                P[dy, b * H1p + h2, b * Hp + 2 * (h2 + dy)] = 1.0
            v2row[b * H1p + h2] = 1.0
    E1 = np.kron(np.eye(cmid, dtype=np.float32), np.ones((CB1, 1), np.float32))
    E2 = np.kron(np.eye(cmid, dtype=np.float32), np.ones((SC2, 1), np.float32))
    E1b = np.kron(np.ones((cmid, 1), np.float32), np.eye(CB1, dtype=np.float32))
    E2b = np.kron(np.ones((cmid, 1), np.float32), np.eye(SC2, dtype=np.float32))
    Ft1 = np.tile(np.eye(CB1, dtype=np.float32), (1, cin))
    Ft2 = np.tile(np.eye(CB1, dtype=np.float32), (1, cmid))
    Fs1 = np.kron(np.eye(cin, dtype=np.float32), np.ones((1, CB1), np.float32))
    Fs2 = np.kron(np.eye(cmid, dtype=np.float32), np.ones((1, CB1), np.float32))
    NT1, NT2 = cin * 4, cmid * 4
    S1 = np.zeros((4, NT1, cin), np.float32)
    S2 = np.zeros((4, NT2, cmid), np.float32)
    for k in range(4):
        for ci in range(cin):
            S1[k, ci * 4 + k, ci] = 1.0
        for ci in range(cmid):
            S2[k, ci * 4 + k, ci] = 1.0
    S1 = S1.reshape(4 * NT1, cin)
    S2 = S2.reshape(4 * NT2, cmid)
    QQ = Q.reshape(2 * CB1, CB1)
    PP = P.reshape(2 * SC2, CB1)
    vc1 = np.tile(vrow1[:, None], (cmid, 1))
    vc2 = np.tile(v2row[:, None], (cmid, 1))
    s1c = np.zeros((W1 - 1, W1p), np.float32)
    s1c[2 * np.arange(W1p), np.arange(W1p)] = 1.0
    s2c = np.zeros((W2 - 1, W2p), np.float32)
    s2c[2 * np.arange(W2p), np.arange(W2p)] = 1.0

    def _pack(parts):
        ncol = max(p.shape[1] for p in parts)
        rows, offs = [], []
        at = 0
        for p in parts:
            r = -(-p.shape[0] // 8) * 8
            q = np.zeros((r, ncol), np.float32)
            q[:p.shape[0], :p.shape[1]] = p
            rows.append(q)
            offs.append(at)
            at += r
        return np.concatenate(rows, 0), offs

    packA, offA = _pack([E1, E2, E1b, E2b, S1, S2, QQ, PP, vc1, vc2])
    packC, offC = _pack([Ft1, Fs1, Ft2, Fs2])
    packD, offD = _pack([s1c, s2c])

    w1r = w_conv1.astype(f32).reshape(cmid, NT1)
    w2r = w_conv2.astype(f32).reshape(cmid, NT2)
    b1r = b_conv1.astype(f32).reshape(cmid, 1)
    b2r = b_conv2.astype(f32).reshape(cmid, 1)

    def body(*refs):
        (x_ref, w1_ref, b1_ref, w2_ref, b2_ref,
         pa_ref, pc_ref, pd_ref) = refs[0:8]
        wf1_refs = refs[8:8 + NS]
        bf1_ref, wf2_ref, bf2_ref, wf3_ref, bf3_ref = refs[8 + NS:13 + NS]
        o_ref = refs[13 + NS]
        xs_sc, fcin_sc, h1_sc = refs[14 + NS:17 + NS]
        i = pl.program_id(0)

        def _a(idx, r, c):
            return pa_ref[offA[idx]:offA[idx] + r, 0:c]

        def dot(a, b):
            return jnp.dot(a, b, preferred_element_type=f32)

        @pl.when(i == 0)
        def _stage1():
            e1_ref = _a(0, M1, cmid)
            e2_ref = _a(1, M2, cmid)
            e1b_ref = _a(2, M1, CB1)
            e2b_ref = _a(3, M2, SC2)
            s1_ref = _a(4, 4 * NT1, cin)
            s2_ref = _a(5, 4 * NT2, cmid)
            q_ref = _a(6, 2 * CB1, CB1)
            p_ref = _a(7, 2 * SC2, CB1)
            vc1_ref = _a(8, M1, 1)
            vc2_ref = _a(9, M2, 1)
            ft1_ref = pc_ref[offC[0]:offC[0] + CB1, 0:cin * CB1]
            fs1_ref = pc_ref[offC[1]:offC[1] + cin, 0:cin * CB1]
            ft2_ref = pc_ref[offC[2]:offC[2] + CB1, 0:cmid * CB1]
            fs2_ref = pc_ref[offC[3]:offC[3] + cmid, 0:cmid * CB1]
            s1c_ref = pd_ref[offD[0]:offD[0] + W1 - 1, 0:W1p]
            s2c_ref = pd_ref[offD[1]:offD[1] + W2 - 1, 0:W2p]
            xs_sc[...] = jnp.zeros((cin * CB1, Wp), f32)
            for ci in range(cin):
                for b in range(B):
                    xs_sc[ci * CB1 + b * Hp + pad:
                          ci * CB1 + b * Hp + pad + H,
                          pad:pad + W] = \
                        x_ref[(b * cin + ci) * H:(b * cin + ci + 1) * H, :]
            xs = xs_sc[...]
            wb1 = dot(e1_ref[...], w1_ref[...])
            msk1 = [dot(dot(e1b_ref[...], q_ref[dy * CB1:(dy + 1) * CB1, :]),
                        ft1_ref[...]) for dy in range(2)]
            a1 = []
            for dx in range(2):
                a = sum(dot(dot(wb1, s1_ref[(dy * 2 + dx) * NT1:
                                            (dy * 2 + dx + 1) * NT1, :]),
                            fs1_ref[...]) * msk1[dy] for dy in range(2))
                a1.append(a)
            b1c = dot(e1_ref[...], b1_ref[...]) * vc1_ref[...]
            c1 = jnp.maximum(dot(a1[0], xs[:, 0:W1])
                             + dot(a1[1], xs[:, 1:Wp]) + b1c, 0.0)
            mw1 = jnp.maximum(c1[:, 0:W1 - 1], c1[:, 1:W1])
            p1c = dot(mw1, s1c_ref[...])
            mh1 = jnp.maximum(
                p1c, jnp.concatenate([p1c[1:, :], p1c[M1 - 1:, :]], axis=0))
            wb2 = dot(e2_ref[...], w2_ref[...])
            msk2 = [dot(dot(e2b_ref[...], p_ref[dy * SC2:(dy + 1) * SC2, :]),
                        ft2_ref[...]) for dy in range(2)]
            a2 = []
            for dx in range(2):
                a = sum(dot(dot(wb2, s2_ref[(dy * 2 + dx) * NT2:
                                            (dy * 2 + dx + 1) * NT2, :]),
                            fs2_ref[...]) * msk2[dy] for dy in range(2))
                a2.append(a)
            b2c = dot(e2_ref[...], b2_ref[...]) * vc2_ref[...]
            c2 = jnp.maximum(dot(a2[0], mh1[:, 0:W2])
                             + dot(a2[1], mh1[:, 1:W1p]) + b2c, 0.0)
            mw2 = jnp.maximum(c2[:, 0:W2 - 1], c2[:, 1:W2])
            p2c = dot(mw2, s2c_ref[...])
            mh2 = jnp.maximum(p2c[:-1, :], p2c[1:, :])
            for c in range(cmid):
                for y in range(H2p):
                    t = c * H2p + y
                    for b in range(B):
                        r = c * SC2 + b * H1p + 2 * y
                        fcin_sc[b:b + 1, t * W2p:(t + 1) * W2p] = \
                            mh2[r:r + 1, :]
            h1_sc[...] = jnp.zeros((SLAB, n_h1), f32)

        h1_sc[...] += sum(
            dot(fcin_sc[:, pl.ds((i * NS + k) * KB, KB)], wf1_refs[k][...])
            for k in range(NS))

        @pl.when(i == G - 1)
        def _stage3():
            h1 = jnp.maximum(h1_sc[...] + bf1_ref[...], 0.0)
            h2 = jnp.maximum(dot(h1, wf2_ref[...]) + bf2_ref[...], 0.0)
            o = dot(h2, wf3_ref[...]) + bf3_ref[...]
            o_ref[...] = o[0:B, :].astype(o_ref.dtype)

    args = (
        x.reshape(B * cin * H, W), w1r, b1r, w2r, b2r,
        jnp.asarray(packA), jnp.asarray(packC), jnp.asarray(packD),
        *([w_fc1.astype(f32)] * NS),
        b_fc1.astype(f32).reshape(1, -1),
        w_fc2.astype(f32), b_fc2.astype(f32).reshape(1, -1),
        w_fc3.astype(f32), b_fc3.astype(f32).reshape(1, -1),
    )
    in_specs = [_cfull(a.shape) for a in args]
    for k in range(NS):
        in_specs[8 + k] = pl.BlockSpec(
            (KB, n_h1), lambda i, _k=k: (i * NS + _k, 0))
    return pl.pallas_call(
        body,
        out_shape=jax.ShapeDtypeStruct((B, n_out), f32),
        grid=(G,),
        in_specs=in_specs,
        out_specs=_cfull((B, n_out)),
        scratch_shapes=[
            pltpu.VMEM((cin * CB1, Wp), f32),
            pltpu.VMEM((SLAB, feat), f32),
            pltpu.VMEM((SLAB, n_h1), f32),
        ],
        compiler_params=pltpu.CompilerParams(
            dimension_semantics=("arbitrary",)),
    )(*args)


# speedup vs baseline: 1.0335x; 1.0335x over previous
import numpy as np
import jax
import jax.numpy as jnp
from jax.experimental import pallas as pl
from jax.experimental.pallas import tpu as pltpu


def _cfull(shape):
    n = len(shape)
    return pl.BlockSpec(tuple(shape), lambda i, _n=n: (0,) * _n)


def kernel(x, w_conv1, b_conv1, w_conv2, b_conv2,
           w_fc1, b_fc1, w_fc2, b_fc2, w_fc3, b_fc3):
    f32 = jnp.float32
    x = x.astype(f32)
    B, cin, H, W = x.shape
    cmid = w_conv1.shape[0]

    pad = 2
    Hp, Wp = H + 2 * pad, W + 2 * pad
    H1, W1 = Hp - 1, Wp - 1
    H1p, W1p = H1 // 2, W1 // 2
    H2, W2 = H1p - 1, W1p - 1
    H2p, W2p = H2 // 2, W2 // 2
    feat = cmid * H2p * W2p
    n_h1, n_h2, n_out = w_fc1.shape[1], w_fc2.shape[1], w_fc3.shape[1]

    CB1 = B * Hp
    M1 = cmid * CB1
    SC2 = 16
    M2 = cmid * SC2
    SLAB = 8
    n_slab = cmid * H2p
    assert B * H1p <= SC2 and B <= SLAB

    NS = 4
    G = 2
    KB = feat // (G * NS)
    assert feat % (G * NS) == 0 and KB % 128 == 0

    Q = np.zeros((2, CB1, CB1), np.float32)
    vrow1 = np.zeros((CB1,), np.float32)
    for b in range(B):
        for h in range(H1):
            for dy in range(2):
                Q[dy, b * Hp + h, b * Hp + h + dy] = 1.0
            vrow1[b * Hp + h] = 1.0
    P = np.zeros((2, SC2, CB1), np.float32)
    v2row = np.zeros((SC2,), np.float32)
    for b in range(B):
        for h2 in range(H2):
            for dy in range(2):
                P[dy, b * H1p + h2, b * Hp + 2 * (h2 + dy)] = 1.0
            v2row[b * H1p + h2] = 1.0
    E1 = np.kron(np.eye(cmid, dtype=np.float32), np.ones((CB1, 1), np.float32))
    E2 = np.kron(np.eye(cmid, dtype=np.float32), np.ones((SC2, 1), np.float32))
    E1b = np.kron(np.ones((cmid, 1), np.float32), np.eye(CB1, dtype=np.float32))
    E2b = np.kron(np.ones((cmid, 1), np.float32), np.eye(SC2, dtype=np.float32))
    Ft1 = np.tile(np.eye(CB1, dtype=np.float32), (1, cin))
    Ft2 = np.tile(np.eye(CB1, dtype=np.float32), (1, cmid))
    Fs1 = np.kron(np.eye(cin, dtype=np.float32), np.ones((1, CB1), np.float32))
    Fs2 = np.kron(np.eye(cmid, dtype=np.float32), np.ones((1, CB1), np.float32))
    NT1, NT2 = cin * 4, cmid * 4
    S1 = np.zeros((4, NT1, cin), np.float32)
    S2 = np.zeros((4, NT2, cmid), np.float32)
    for k in range(4):
        for ci in range(cin):
            S1[k, ci * 4 + k, ci] = 1.0
        for ci in range(cmid):
            S2[k, ci * 4 + k, ci] = 1.0
    S1 = S1.reshape(4 * NT1, cin)
    S2 = S2.reshape(4 * NT2, cmid)
    QQ = Q.reshape(2 * CB1, CB1)
    PP = P.reshape(2 * SC2, CB1)
    vc1 = np.tile(vrow1[:, None], (cmid, 1))
    vc2 = np.tile(v2row[:, None], (cmid, 1))
    s1c = np.zeros((W1 - 1, W1p), np.float32)
    s1c[2 * np.arange(W1p), np.arange(W1p)] = 1.0
    s2c = np.zeros((W2 - 1, W2p), np.float32)
    s2c[2 * np.arange(W2p), np.arange(W2p)] = 1.0

    def _pack(parts):
        ncol = max(p.shape[1] for p in parts)
        rows, offs = [], []
        at = 0
        for p in parts:
            r = -(-p.shape[0] // 8) * 8
            q = np.zeros((r, ncol), np.float32)
            q[:p.shape[0], :p.shape[1]] = p
            rows.append(q)
            offs.append(at)
            at += r
        return np.concatenate(rows, 0), offs

    packA, offA = _pack([E1, E2, E1b, E2b, S1, S2, QQ, PP, vc1, vc2])
    packC, offC = _pack([Ft1, Fs1, Ft2, Fs2])
    packD, offD = _pack([s1c, s2c])

    w1r = w_conv1.astype(f32).reshape(cmid, NT1)
    w2r = w_conv2.astype(f32).reshape(cmid, NT2)
    b1r = b_conv1.astype(f32).reshape(cmid, 1)
    b2r = b_conv2.astype(f32).reshape(cmid, 1)

    def body(*refs):
        (x_ref, w1_ref, b1_ref, w2_ref, b2_ref,
         pa_ref, pc_ref, pd_ref) = refs[0:8]
        wf1_refs = refs[8:8 + NS]
        bf1_ref, wf2_ref, bf2_ref, wf3_ref, bf3_ref = refs[8 + NS:13 + NS]
        o_ref = refs[13 + NS]
        xs_sc, fcin_sc, h1_sc = refs[14 + NS:17 + NS]
        i = pl.program_id(0)

        def _a(idx, r, c):
            return pa_ref[offA[idx]:offA[idx] + r, 0:c]

        def dot(a, b):
            return jnp.dot(a, b, preferred_element_type=f32)

        @pl.when(i == 0)
        def _stage1():
            e1_ref = _a(0, M1, cmid)
            e2_ref = _a(1, M2, cmid)
            e1b_ref = _a(2, M1, CB1)
            e2b_ref = _a(3, M2, SC2)
            s1_ref = _a(4, 4 * NT1, cin)
            s2_ref = _a(5, 4 * NT2, cmid)
            q_ref = _a(6, 2 * CB1, CB1)
            p_ref = _a(7, 2 * SC2, CB1)
            vc1_ref = _a(8, M1, 1)
            vc2_ref = _a(9, M2, 1)
            ft1_ref = pc_ref[offC[0]:offC[0] + CB1, 0:cin * CB1]
            fs1_ref = pc_ref[offC[1]:offC[1] + cin, 0:cin * CB1]
            ft2_ref = pc_ref[offC[2]:offC[2] + CB1, 0:cmid * CB1]
            fs2_ref = pc_ref[offC[3]:offC[3] + cmid, 0:cmid * CB1]
            s1c_ref = pd_ref[offD[0]:offD[0] + W1 - 1, 0:W1p]
            s2c_ref = pd_ref[offD[1]:offD[1] + W2 - 1, 0:W2p]
            xs_sc[...] = jnp.zeros((cin * CB1, Wp), f32)
            for ci in range(cin):
                for b in range(B):
                    xs_sc[ci * CB1 + b * Hp + pad:
                          ci * CB1 + b * Hp + pad + H,
                          pad:pad + W] = \
                        x_ref[(b * cin + ci) * H:(b * cin + ci + 1) * H, :]
            xs = xs_sc[...]
            wb1 = dot(e1_ref[...], w1_ref[...])
            msk1 = [dot(dot(e1b_ref[...], q_ref[dy * CB1:(dy + 1) * CB1, :]),
                        ft1_ref[...]) for dy in range(2)]
            a1 = []
            for dx in range(2):
                a = sum(dot(dot(wb1, s1_ref[(dy * 2 + dx) * NT1:
                                            (dy * 2 + dx + 1) * NT1, :]),
                            fs1_ref[...]) * msk1[dy] for dy in range(2))
                a1.append(a)
            b1c = dot(e1_ref[...], b1_ref[...]) * vc1_ref[...]
            c1 = jnp.maximum(dot(a1[0], xs[:, 0:W1])
                             + dot(a1[1], xs[:, 1:Wp]) + b1c, 0.0)
            mw1 = jnp.maximum(c1[:, 0:W1 - 1], c1[:, 1:W1])
            p1c = dot(mw1, s1c_ref[...])
            mh1 = jnp.maximum(
                p1c, jnp.concatenate([p1c[1:, :], p1c[M1 - 1:, :]], axis=0))
            wb2 = dot(e2_ref[...], w2_ref[...])
            msk2 = [dot(dot(e2b_ref[...], p_ref[dy * SC2:(dy + 1) * SC2, :]),
                        ft2_ref[...]) for dy in range(2)]
            a2 = []
            for dx in range(2):
                a = sum(dot(dot(wb2, s2_ref[(dy * 2 + dx) * NT2:
                                            (dy * 2 + dx + 1) * NT2, :]),
                            fs2_ref[...]) * msk2[dy] for dy in range(2))
                a2.append(a)
            b2c = dot(e2_ref[...], b2_ref[...]) * vc2_ref[...]
            c2 = jnp.maximum(dot(a2[0], mh1[:, 0:W2])
                             + dot(a2[1], mh1[:, 1:W1p]) + b2c, 0.0)
            mw2 = jnp.maximum(c2[:, 0:W2 - 1], c2[:, 1:W2])
            p2c = dot(mw2, s2c_ref[...])
            mh2 = jnp.maximum(p2c[:-1, :], p2c[1:, :])
            for c in range(cmid):
                for y in range(H2p):
                    t = c * H2p + y
                    for b in range(B):
                        r = c * SC2 + b * H1p + 2 * y
                        fcin_sc[b:b + 1, t * W2p:(t + 1) * W2p] = \
                            mh2[r:r + 1, :]
            h1_sc[...] = jnp.zeros((SLAB, n_h1), f32)

        h1_sc[...] += sum(
            dot(fcin_sc[:, pl.ds((i * NS + k) * KB, KB)], wf1_refs[k][...])
            for k in range(NS))

        @pl.when(i == G - 1)
        def _stage3():
            h1 = jnp.maximum(h1_sc[...] + bf1_ref[...], 0.0)
            h2 = jnp.maximum(dot(h1, wf2_ref[...]) + bf2_ref[...], 0.0)
            o = dot(h2, wf3_ref[...]) + bf3_ref[...]
            o_ref[...] = o[0:B, :].astype(o_ref.dtype)

    args = (
        x.reshape(B * cin * H, W), w1r, b1r, w2r, b2r,
        jnp.asarray(packA), jnp.asarray(packC), jnp.asarray(packD),
        *([w_fc1.astype(f32)] * NS),
        b_fc1.astype(f32).reshape(1, -1),
        w_fc2.astype(f32), b_fc2.astype(f32).reshape(1, -1),
        w_fc3.astype(f32), b_fc3.astype(f32).reshape(1, -1),
    )
    in_specs = [_cfull(a.shape) for a in args]
    for k in range(NS):
        in_specs[8 + k] = pl.BlockSpec(
            (KB, n_h1), lambda i, _k=k: (i * NS + _k, 0))
    return pl.pallas_call(
        body,
        out_shape=jax.ShapeDtypeStruct((B, n_out), f32),
        grid=(G,),
        in_specs=in_specs,
        out_specs=_cfull((B, n_out)),
        scratch_shapes=[
            pltpu.VMEM((cin * CB1, Wp), f32),
            pltpu.VMEM((SLAB, feat), f32),
            pltpu.VMEM((SLAB, n_h1), f32),
        ],
        compiler_params=pltpu.CompilerParams(
            dimension_semantics=("arbitrary",)),
    )(*args)
